# Initial kernel scaffold; baseline (speedup 1.0000x reference)
import jax
import jax.numpy as jnp
from jax.experimental import pallas as pl


def kernel(indices, tables):
    raise NotImplementedError("write your pallas kernel here")


# speedup vs baseline: 15.6140x; 15.6140x over previous
import functools

import jax
import jax.numpy as jnp
from jax import lax
from jax.experimental import pallas as pl
from jax.experimental.pallas import tpu as pltpu
from jax.experimental.pallas import tpu_sc as plsc

_NUM_TABLES = 26
_VOCAB = 100000
_EMB = 32
_BATCH = 16384
_HIST = 20

_NC = 2
_NS = 16
_NW = _NC * _NS
_B_PER_W = _BATCH // _NW
_CHUNK = 64
_N_CHUNK = _B_PER_W // _CHUNK
_IDX_PER_CHUNK = _CHUNK * _HIST
_GROUPS = [(0, 4), (4, 4), (8, 4), (12, 4), (16, 4), (20, 4), (24, 2)]


def _sc_embedding_bag(idx_flat, tab_flat):
    mesh = plsc.VectorSubcoreMesh(core_axis_name="c", subcore_axis_name="s")

    @functools.partial(
        pl.kernel,
        mesh=mesh,
        compiler_params=pltpu.CompilerParams(use_tc_tiling_on_sc=False),
        out_type=jax.ShapeDtypeStruct((_BATCH, _NUM_TABLES * _EMB), jnp.float32),
        scratch_types=[
            pltpu.VMEM((_IDX_PER_CHUNK,), jnp.int32),
            pltpu.VMEM((_IDX_PER_CHUNK, _EMB), jnp.float32),
            pltpu.VMEM((_CHUNK, 4 * _EMB), jnp.float32),
            pltpu.SemaphoreType.DMA,
        ],
    )
    def k(idx_hbm, tab_hbm, out_hbm, idx_v, rows_v, acc_v, sem):
        wid = lax.axis_index("s") * _NC + lax.axis_index("c")
        b0 = wid * _B_PER_W

        for t0, gw in _GROUPS:

            def chunk_body(c, _, t0=t0, gw=gw):
                bstart = b0 + c * _CHUNK

                for tl in range(gw):
                    t = t0 + tl
                    toff = t * _VOCAB
                    src_off = t * (_BATCH * _HIST) + bstart * _HIST
                    pltpu.sync_copy(
                        idx_hbm.at[pl.ds(src_off, _IDX_PER_CHUNK)], idx_v)

                    def add_off(i, _, toff=toff):
                        sl = pl.ds(i * 16, 16)
                        idx_v[sl] = idx_v[sl] + toff
                        return 0

                    lax.fori_loop(0, _IDX_PER_CHUNK // 16, add_off, 0)

                    pltpu.async_copy(tab_hbm.at[idx_v], rows_v, sem).wait()

                    def bag(j, _, tl=tl):
                        base = j * _HIST
                        lo = rows_v[base, 0:16]
                        hi = rows_v[base, 16:32]
                        for h in range(1, _HIST):
                            lo = lo + rows_v[base + h, 0:16]
                            hi = hi + rows_v[base + h, 16:32]
                        acc_v[j, pl.ds(tl * _EMB, 16)] = lo
                        acc_v[j, pl.ds(tl * _EMB + 16, 16)] = hi
                        return 0

                    lax.fori_loop(0, _CHUNK, bag, 0)

                pltpu.sync_copy(
                    acc_v.at[:, pl.ds(0, gw * _EMB)],
                    out_hbm.at[pl.ds(bstart, _CHUNK),
                               pl.ds(t0 * _EMB, gw * _EMB)])
                return 0

            lax.fori_loop(0, _N_CHUNK, chunk_body, 0)

    return k(idx_flat, tab_flat)


def kernel(indices, tables):
    idx_flat = indices.reshape(-1)
    tab_flat = tables.reshape(_NUM_TABLES * _VOCAB, _EMB)
    return _sc_embedding_bag(idx_flat, tab_flat)


# speedup vs baseline: 16.0327x; 1.0268x over previous
import functools

import jax
import jax.numpy as jnp
from jax import lax
from jax.experimental import pallas as pl
from jax.experimental.pallas import tpu as pltpu
from jax.experimental.pallas import tpu_sc as plsc

_NUM_TABLES = 26
_VOCAB = 100000
_EMB = 32
_BATCH = 16384
_HIST = 20

_NC = 2
_NS = 16
_NW = _NC * _NS
_B_PER_W = _BATCH // _NW
_CHUNK = 64
_N_CHUNK = _B_PER_W // _CHUNK


def _sc_embedding_bag(indices, tables):
    mesh = plsc.VectorSubcoreMesh(core_axis_name="c", subcore_axis_name="s")

    @functools.partial(
        pl.kernel,
        mesh=mesh,
        compiler_params=pltpu.CompilerParams(use_tc_tiling_on_sc=False),
        out_type=jax.ShapeDtypeStruct((_BATCH, _NUM_TABLES * _EMB), jnp.float32),
        scratch_types=[
            pltpu.VMEM((_CHUNK * _HIST,), jnp.int32),
            pltpu.VMEM((_CHUNK * _HIST, _EMB), jnp.float32),
            pltpu.VMEM((_CHUNK, _EMB), jnp.float32),
            pltpu.SemaphoreType.DMA,
        ],
    )
    def k(idx_hbm, tab_hbm, out_hbm, idx_v, rows_v, acc_v, sem):
        wid = lax.axis_index("s") * _NC + lax.axis_index("c")
        b0 = wid * _B_PER_W

        def table_body(t, _):

            def chunk_body(c, _):
                bstart = b0 + c * _CHUNK
                src_off = t * (_BATCH * _HIST) + bstart * _HIST
                pltpu.sync_copy(
                    idx_hbm.at[pl.ds(src_off, _CHUNK * _HIST)], idx_v)

                pltpu.async_copy(
                    tab_hbm.at[t].at[idx_v], rows_v, sem).wait()

                def bag(j, _):
                    base = j * _HIST
                    lo = rows_v[base, 0:16]
                    hi = rows_v[base, 16:32]
                    for h in range(1, _HIST):
                        lo = lo + rows_v[base + h, 0:16]
                        hi = hi + rows_v[base + h, 16:32]
                    acc_v[j, 0:16] = lo
                    acc_v[j, 16:32] = hi
                    return 0

                lax.fori_loop(0, _CHUNK, bag, 0)

                pltpu.sync_copy(
                    acc_v,
                    out_hbm.at[pl.ds(bstart, _CHUNK), pl.ds(t * _EMB, _EMB)])
                return 0

            lax.fori_loop(0, _N_CHUNK, chunk_body, 0)
            return 0

        lax.fori_loop(0, _NUM_TABLES, table_body, 0)

    return k(indices, tables)


def kernel(indices, tables):
    return _sc_embedding_bag(indices.reshape(-1), tables)
